# Initial kernel scaffold; baseline (speedup 1.0000x reference)
import jax
import jax.numpy as jnp
from jax.experimental import pallas as pl


def kernel(x, Wr, br, gamma, beta, W1, b1, W2, b2):
    raise NotImplementedError("write your pallas kernel here")


# speedup vs baseline: 3.3455x; 3.3455x over previous
import functools
import math

import jax
import jax.numpy as jnp
from jax.experimental import pallas as pl
from jax.experimental.pallas import tpu as pltpu

_E = 8
_H = 128
_F = 256
_O = 64
_D = _E * _H
_TILE = 512
_SQRT2 = math.sqrt(2.0)


def _moe_body(x_ref, wr_ref, br_ref, gamma_ref, beta_ref, w1_ref, b1_ref,
              w2_ref, b2_ref, o_ref):
    xt = x_ref[:, :]
    logits = jnp.dot(xt, wr_ref[:, :],
                     preferred_element_type=jnp.float32) + br_ref[0, :]
    mu = jnp.mean(logits, axis=-1, keepdims=True)
    var = jnp.mean((logits - mu) ** 2, axis=-1, keepdims=True)
    normed = ((logits - mu) / jnp.sqrt(var + 1e-5)) * gamma_ref[0, :] \
        + beta_ref[0, :]
    m = jnp.max(normed, axis=-1, keepdims=True)
    ex = jnp.exp(normed - m)
    router = ex / jnp.sum(ex, axis=-1, keepdims=True)

    cols = []
    for e in range(_E):
        he = jnp.dot(xt[:, e * _H:(e + 1) * _H], w1_ref[e],
                     preferred_element_type=jnp.float32) + b1_ref[e]
        ge = 0.5 * he * (1.0 + jax.lax.erf(he / _SQRT2))
        cols.append(router[:, e:e + 1] * ge)
    gmat = jnp.concatenate(cols, axis=1)

    y = jnp.dot(gmat, w2_ref[:, :], preferred_element_type=jnp.float32)
    y = y + jnp.dot(router, b2_ref[:, :],
                    preferred_element_type=jnp.float32)
    o_ref[:, :] = y


@functools.partial(jax.jit, static_argnames=())
def kernel(x, Wr, br, gamma, beta, W1, b1, W2, b2):
    B, T, D = x.shape
    BT = B * T
    xf = x.reshape(BT, D)
    w2f = W2.reshape(_E * _F, _O)
    grid = (BT // _TILE,)

    out = pl.pallas_call(
        _moe_body,
        grid=grid,
        in_specs=[
            pl.BlockSpec((_TILE, D), lambda i: (i, 0)),
            pl.BlockSpec((D, _E), lambda i: (0, 0)),
            pl.BlockSpec((1, _E), lambda i: (0, 0)),
            pl.BlockSpec((1, _E), lambda i: (0, 0)),
            pl.BlockSpec((1, _E), lambda i: (0, 0)),
            pl.BlockSpec((_E, _H, _F), lambda i: (0, 0, 0)),
            pl.BlockSpec((_E, _F), lambda i: (0, 0)),
            pl.BlockSpec((_E * _F, _O), lambda i: (0, 0)),
            pl.BlockSpec((_E, _O), lambda i: (0, 0)),
        ],
        out_specs=pl.BlockSpec((_TILE, _O), lambda i: (i, 0)),
        out_shape=jax.ShapeDtypeStruct((BT, _O), jnp.float32),
        compiler_params=pltpu.CompilerParams(
            dimension_semantics=("parallel",),
        ),
    )(xf, Wr, br.reshape(1, _E), gamma.reshape(1, _E), beta.reshape(1, _E),
      W1, b1, w2f, b2)
    return out.reshape(B, T, _O)
